# baseline (speedup 1.0000x reference)
import functools

import jax
import jax.numpy as jnp
from jax import lax
from jax.experimental import pallas as pl
from jax.experimental.pallas import tpu as pltpu
from jax.experimental.pallas import tpu_sc as plsc

_NUM_CORES = 2
_NUM_SUBCORES = 16
_NW = _NUM_CORES * _NUM_SUBCORES
_L = 16


def _transpose_kernel(N: int, D: int):
    n_full = N // 128
    rem = N - n_full * 128
    n_vis = (n_full + _NW - 1) // _NW
    mesh = plsc.VectorSubcoreMesh(core_axis_name="c", subcore_axis_name="s")

    @functools.partial(
        pl.kernel,
        mesh=mesh,
        compiler_params=pltpu.CompilerParams(use_tc_tiling_on_sc=True, needs_layout_passes=False),
        out_type=jax.ShapeDtypeStruct((N, 128), jnp.float32),
        scratch_types=[pltpu.VMEM((D, 128), jnp.float32)] * 2
        + [pltpu.VMEM((128, 128), jnp.float32)] * 2
        + [pltpu.VMEM((128 * D,), jnp.float32)]
        + [pltpu.SemaphoreType.DMA] * 4,
    )
    def k(tT, tail, rows, vin0, vin1, vout0, vout1, vtail, r0, r1, w0, w1):
        vins = (vin0, vin1)
        vouts = (vout0, vout1)
        rsems = (r0, r1)
        wsems = (w0, w1)
        wid = lax.axis_index("s") * _NUM_CORES + lax.axis_index("c")
        iota = lax.iota(jnp.int32, _L)

        def panel_of(v):
            return wid + v * _NW

        def fire_read(p, s):
            pltpu.async_copy(
                tT.at[:, pl.ds(pl.multiple_of(p * 128, 128), 128)],
                vins[s], rsems[s])

        def drain_read(s):
            pltpu.make_async_copy(
                tT.at[:, pl.ds(0, 128)], vins[s], rsems[s]).wait()

        def fire_write(p, s):
            pltpu.async_copy(
                vouts[s],
                rows.at[pl.ds(pl.multiple_of(p * 128, 8), 128)], wsems[s])

        def drain_write(s):
            pltpu.make_async_copy(
                vouts[s], rows.at[pl.ds(0, 128)], wsems[s]).wait()

        rots = [(iota + d) % _L for d in range(_L)]

        def transpose_panel(s):
            def blk(i8, carry):
                ivec = i8 * _L + iota
                for c0 in range(0, D, _L):
                    for d in range(_L):
                        cd = c0 + rots[d]
                        vals = plsc.load_gather(vins[s], [cd, ivec])
                        plsc.store_scatter(vouts[s], [ivec, cd], vals)
                return carry
            lax.fori_loop(0, 8, blk, 0, unroll=4)

        @pl.when(panel_of(0) < n_full)
        def _():
            fire_read(panel_of(0), 0)

        def visit(v, s):
            p = panel_of(v)

            @pl.when(p < n_full)
            def _():
                pn = panel_of(v + 1)

                @pl.when(pn < n_full)
                def _():
                    fire_read(pn, 1 - s)

                drain_read(s)

                @pl.when(v >= 2)
                def _():
                    drain_write(s)

                transpose_panel(s)
                fire_write(p, s)

        def body(i, carry):
            visit(2 * i, 0)
            visit(2 * i + 1, 1)
            return carry

        lax.fori_loop(0, (n_vis + 1) // 2, body, 0, unroll=False)
        drain_write(0)
        drain_write(1)

        @pl.when(wid == 0)
        def _():
            pltpu.sync_copy(tail, vtail.at[pl.ds(0, rem * D)])

            def row(i, carry):
                for c0 in range(0, D, _L):
                    vout0.at[i][pl.ds(c0, _L)] = vtail[pl.ds(i * D + c0, _L)]
                return carry
            lax.fori_loop(0, rem, row, 0, unroll=4)
            pltpu.sync_copy(vout0.at[pl.ds(0, rem)],
                            rows.at[pl.ds(n_full * 128, rem)])

    return k


def _gather_kernel(N: int, D: int, B: int):
    b_per_w = B // _NW
    n_pan = b_per_w // 128
    mesh = plsc.VectorSubcoreMesh(core_axis_name="c", subcore_axis_name="s")

    @functools.partial(
        pl.kernel,
        mesh=mesh,
        compiler_params=pltpu.CompilerParams(
            use_tc_tiling_on_sc=True, needs_layout_passes=False),
        out_type=jax.ShapeDtypeStruct((D, B), jnp.float32),
        scratch_types=[pltpu.VMEM((b_per_w,), jnp.int32)]
        + [pltpu.VMEM((128, 128), jnp.float32)] * 2
        + [pltpu.VMEM((D, 128), jnp.float32)] * 2
        + [pltpu.SemaphoreType.DMA] * 4,
    )
    def k(rows, idx_hbm, outT, idx_v, vr0, vr1, vp0, vp1, g0, g1, w0, w1):
        vrows = (vr0, vr1)
        vpans = (vp0, vp1)
        gsems = (g0, g1)
        wsems = (w0, w1)
        wid = lax.axis_index("s") * _NUM_CORES + lax.axis_index("c")
        iota = lax.iota(jnp.int32, _L)
        rots = [(iota + d) % _L for d in range(_L)]
        pltpu.sync_copy(idx_hbm.at[pl.ds(wid * b_per_w, b_per_w)], idx_v)
        col0 = wid * n_pan

        def fire_gather(g, s):
            pltpu.async_copy(
                rows.at[idx_v.at[pl.ds(g * 128, 128)]], vrows[s], gsems[s])

        def drain_gather(s):
            pltpu.make_async_copy(
                rows.at[pl.ds(0, 128)], vrows[s], gsems[s]).wait()

        def fire_write(g, s):
            pltpu.async_copy(
                vpans[s],
                outT.at[:, pl.ds(pl.multiple_of((col0 + g) * 128, 128), 128)],
                wsems[s])

        def drain_write(s):
            pltpu.make_async_copy(
                vpans[s], outT.at[:, pl.ds(0, 128)], wsems[s]).wait()

        def assemble(s):
            def blk(i8, carry):
                ivec = i8 * _L + iota
                for c0 in range(0, D, _L):
                    for d in range(_L):
                        cd = c0 + rots[d]
                        vals = plsc.load_gather(vrows[s], [ivec, cd])
                        plsc.store_scatter(vpans[s], [cd, ivec], vals)
                return carry
            lax.fori_loop(0, 8, blk, 0, unroll=4)

        fire_gather(0, 0)

        def visit(g, s):
            @pl.when(g + 1 < n_pan)
            def _():
                fire_gather(g + 1, 1 - s)
            drain_gather(s)

            @pl.when(g >= 2)
            def _():
                drain_write(s)
            assemble(s)
            fire_write(g, s)

        def body(i, carry):
            visit(2 * i, 0)
            visit(2 * i + 1, 1)
            return carry

        lax.fori_loop(0, n_pan // 2, body, 0, unroll=False)
        for s in (0, 1):
            drain_write(s)

    return k


def kernel(embedding, idx_mapping):
    B = idx_mapping.shape[0]
    N, D = embedding.shape
    tT = embedding.T
    tail = embedding[(N // 128) * 128:].reshape(-1)
    rows = _transpose_kernel(N, D)(tT, tail)
    outT = _gather_kernel(N, D, B)(rows, idx_mapping.astype(jnp.int32))
    return outT.T


# speedup vs baseline: 1.0874x; 1.0874x over previous
import functools

import jax
import jax.numpy as jnp
from jax import lax
from jax.experimental import pallas as pl
from jax.experimental.pallas import tpu as pltpu
from jax.experimental.pallas import tpu_sc as plsc

_NUM_CORES = 2
_NUM_SUBCORES = 16
_NW = _NUM_CORES * _NUM_SUBCORES
_L = 16


def _transpose_kernel(N: int, D: int):
    n_full = N // 128
    rem = N - n_full * 128
    n_vis = (n_full + _NW - 1) // _NW
    mesh = plsc.VectorSubcoreMesh(core_axis_name="c", subcore_axis_name="s")

    @functools.partial(
        pl.kernel,
        mesh=mesh,
        compiler_params=pltpu.CompilerParams(use_tc_tiling_on_sc=True, needs_layout_passes=False),
        out_type=jax.ShapeDtypeStruct((N, 128), jnp.float32),
        scratch_types=[pltpu.VMEM((D, 128), jnp.float32)] * 2
        + [pltpu.VMEM((128, 128), jnp.float32)] * 2
        + [pltpu.VMEM((128 * D,), jnp.float32)]
        + [pltpu.SemaphoreType.DMA] * 4,
    )
    def k(tT, tail, rows, vin0, vin1, vout0, vout1, vtail, r0, r1, w0, w1):
        vins = (vin0, vin1)
        vouts = (vout0, vout1)
        rsems = (r0, r1)
        wsems = (w0, w1)
        wid = lax.axis_index("s") * _NUM_CORES + lax.axis_index("c")
        iota = lax.iota(jnp.int32, _L)

        def panel_of(v):
            return wid + v * _NW

        def fire_read(p, s):
            pltpu.async_copy(
                tT.at[:, pl.ds(pl.multiple_of(p * 128, 128), 128)],
                vins[s], rsems[s])

        def drain_read(s):
            pltpu.make_async_copy(
                tT.at[:, pl.ds(0, 128)], vins[s], rsems[s]).wait()

        def fire_write(p, s):
            pltpu.async_copy(
                vouts[s],
                rows.at[pl.ds(pl.multiple_of(p * 128, 8), 128)], wsems[s])

        def drain_write(s):
            pltpu.make_async_copy(
                vouts[s], rows.at[pl.ds(0, 128)], wsems[s]).wait()

        rots = [(iota + d) % _L for d in range(_L)]

        def transpose_panel(s):
            def blk(i8, carry):
                ivec = i8 * _L + iota
                for c0 in range(0, D, _L):
                    for d in range(_L):
                        cd = c0 + rots[d]
                        vals = plsc.load_gather(vins[s], [cd, ivec])
                        plsc.store_scatter(vouts[s], [ivec, cd], vals)
                return carry
            lax.fori_loop(0, 8, blk, 0, unroll=2)

        @pl.when(panel_of(0) < n_full)
        def _():
            fire_read(panel_of(0), 0)

        def visit(v, s):
            p = panel_of(v)

            @pl.when(p < n_full)
            def _():
                pn = panel_of(v + 1)

                @pl.when(pn < n_full)
                def _():
                    fire_read(pn, 1 - s)

                drain_read(s)

                @pl.when(v >= 2)
                def _():
                    drain_write(s)

                transpose_panel(s)
                fire_write(p, s)

        def body(i, carry):
            visit(2 * i, 0)
            visit(2 * i + 1, 1)
            return carry

        lax.fori_loop(0, (n_vis + 1) // 2, body, 0, unroll=False)
        drain_write(0)
        drain_write(1)

        @pl.when(wid == 0)
        def _():
            pltpu.sync_copy(tail, vtail.at[pl.ds(0, rem * D)])

            def row(i, carry):
                for c0 in range(0, D, _L):
                    vout0.at[i][pl.ds(c0, _L)] = vtail[pl.ds(i * D + c0, _L)]
                return carry
            lax.fori_loop(0, rem, row, 0, unroll=4)
            pltpu.sync_copy(vout0.at[pl.ds(0, rem)],
                            rows.at[pl.ds(n_full * 128, rem)])

    return k


def _gather_kernel(N: int, D: int, B: int):
    b_per_w = B // _NW
    n_pan = b_per_w // 128
    mesh = plsc.VectorSubcoreMesh(core_axis_name="c", subcore_axis_name="s")

    @functools.partial(
        pl.kernel,
        mesh=mesh,
        compiler_params=pltpu.CompilerParams(
            use_tc_tiling_on_sc=True, needs_layout_passes=False),
        out_type=jax.ShapeDtypeStruct((D, B), jnp.float32),
        scratch_types=[pltpu.VMEM((b_per_w,), jnp.int32)]
        + [pltpu.VMEM((128, 128), jnp.float32)] * 2
        + [pltpu.VMEM((D, 128), jnp.float32)] * 2
        + [pltpu.SemaphoreType.DMA] * 4,
    )
    def k(rows, idx_hbm, outT, idx_v, vr0, vr1, vp0, vp1, g0, g1, w0, w1):
        vrows = (vr0, vr1)
        vpans = (vp0, vp1)
        gsems = (g0, g1)
        wsems = (w0, w1)
        wid = lax.axis_index("s") * _NUM_CORES + lax.axis_index("c")
        iota = lax.iota(jnp.int32, _L)
        rots = [(iota + d) % _L for d in range(_L)]
        pltpu.sync_copy(idx_hbm.at[pl.ds(wid * b_per_w, b_per_w)], idx_v)
        col0 = wid * n_pan

        def fire_gather(g, s):
            pltpu.async_copy(
                rows.at[idx_v.at[pl.ds(g * 128, 128)]], vrows[s], gsems[s])

        def drain_gather(s):
            pltpu.make_async_copy(
                rows.at[pl.ds(0, 128)], vrows[s], gsems[s]).wait()

        def fire_write(g, s):
            pltpu.async_copy(
                vpans[s],
                outT.at[:, pl.ds(pl.multiple_of((col0 + g) * 128, 128), 128)],
                wsems[s])

        def drain_write(s):
            pltpu.make_async_copy(
                vpans[s], outT.at[:, pl.ds(0, 128)], wsems[s]).wait()

        def assemble(s):
            def blk(i8, carry):
                ivec = i8 * _L + iota
                for c0 in range(0, D, _L):
                    for d in range(_L):
                        cd = c0 + rots[d]
                        vals = plsc.load_gather(vrows[s], [ivec, cd])
                        plsc.store_scatter(vpans[s], [cd, ivec], vals)
                return carry
            lax.fori_loop(0, 8, blk, 0, unroll=2)

        fire_gather(0, 0)

        def visit(g, s):
            @pl.when(g + 1 < n_pan)
            def _():
                fire_gather(g + 1, 1 - s)
            drain_gather(s)

            @pl.when(g >= 2)
            def _():
                drain_write(s)
            assemble(s)
            fire_write(g, s)

        def body(i, carry):
            visit(2 * i, 0)
            visit(2 * i + 1, 1)
            return carry

        lax.fori_loop(0, n_pan // 2, body, 0, unroll=False)
        for s in (0, 1):
            drain_write(s)

    return k


def kernel(embedding, idx_mapping):
    B = idx_mapping.shape[0]
    N, D = embedding.shape
    tT = embedding.T
    tail = embedding[(N // 128) * 128:].reshape(-1)
    rows = _transpose_kernel(N, D)(tT, tail)
    outT = _gather_kernel(N, D, B)(rows, idx_mapping.astype(jnp.int32))
    return outT.T


# speedup vs baseline: 1.8638x; 1.7140x over previous
import functools

import jax
import jax.numpy as jnp
from jax import lax
from jax.experimental import pallas as pl
from jax.experimental.pallas import tpu as pltpu
from jax.experimental.pallas import tpu_sc as plsc

_NUM_CORES = 2
_NUM_SUBCORES = 16
_NW = _NUM_CORES * _NUM_SUBCORES
_L = 16


def _transpose_kernel(N: int, D: int):
    n_full = N // 128
    rem = N - n_full * 128
    n_vis = (n_full + _NW - 1) // _NW
    mesh = plsc.VectorSubcoreMesh(core_axis_name="c", subcore_axis_name="s")

    @functools.partial(
        pl.kernel,
        mesh=mesh,
        compiler_params=pltpu.CompilerParams(use_tc_tiling_on_sc=True, needs_layout_passes=False),
        out_type=jax.ShapeDtypeStruct((N, 128), jnp.float32),
        scratch_types=[pltpu.VMEM((D, 128), jnp.float32)] * 2
        + [pltpu.VMEM((128, 128), jnp.float32)] * 2
        + [pltpu.VMEM((128 * D,), jnp.float32)]
        + [pltpu.SemaphoreType.DMA] * 4,
    )
    def k(tT, tail, rows, vin0, vin1, vout0, vout1, vtail, r0, r1, w0, w1):
        vins = (vin0, vin1)
        vouts = (vout0, vout1)
        rsems = (r0, r1)
        wsems = (w0, w1)
        wid = lax.axis_index("s") * _NUM_CORES + lax.axis_index("c")
        iota = lax.iota(jnp.int32, _L)

        def panel_of(v):
            return wid + v * _NW

        def fire_read(p, s):
            pltpu.async_copy(
                tT.at[:, pl.ds(pl.multiple_of(p * 128, 128), 128)],
                vins[s], rsems[s])

        def drain_read(s):
            pltpu.make_async_copy(
                tT.at[:, pl.ds(0, 128)], vins[s], rsems[s]).wait()

        def fire_write(p, s):
            pltpu.async_copy(
                vouts[s],
                rows.at[pl.ds(pl.multiple_of(p * 128, 8), 128)], wsems[s])

        def drain_write(s):
            pltpu.make_async_copy(
                vouts[s], rows.at[pl.ds(0, 128)], wsems[s]).wait()

        rots = [(iota + d) % _L for d in range(_L)]

        def transpose_panel(s):
            def blk(i8, carry):
                ivec = i8 * _L + iota
                for c0 in range(0, D, _L):
                    cds = [c0 + rots[d] for d in range(_L)]
                    vals = [plsc.load_gather(vins[s], [cd, ivec]) for cd in cds]
                    for cd, v in zip(cds, vals):
                        plsc.store_scatter(vouts[s], [ivec, cd], v)
                return carry
            lax.fori_loop(0, 8, blk, 0, unroll=2)

        @pl.when(panel_of(0) < n_full)
        def _():
            fire_read(panel_of(0), 0)

        def visit(v, s):
            p = panel_of(v)

            @pl.when(p < n_full)
            def _():
                pn = panel_of(v + 1)

                @pl.when(pn < n_full)
                def _():
                    fire_read(pn, 1 - s)

                drain_read(s)

                @pl.when(v >= 2)
                def _():
                    drain_write(s)

                transpose_panel(s)
                fire_write(p, s)

        def body(i, carry):
            visit(2 * i, 0)
            visit(2 * i + 1, 1)
            return carry

        lax.fori_loop(0, (n_vis + 1) // 2, body, 0, unroll=False)
        drain_write(0)
        drain_write(1)

        @pl.when(wid == 0)
        def _():
            pltpu.sync_copy(tail, vtail.at[pl.ds(0, rem * D)])

            def row(i, carry):
                for c0 in range(0, D, _L):
                    vout0.at[i][pl.ds(c0, _L)] = vtail[pl.ds(i * D + c0, _L)]
                return carry
            lax.fori_loop(0, rem, row, 0, unroll=4)
            pltpu.sync_copy(vout0.at[pl.ds(0, rem)],
                            rows.at[pl.ds(n_full * 128, rem)])

    return k


def _gather_kernel(N: int, D: int, B: int):
    b_per_w = B // _NW
    n_pan = b_per_w // 128
    mesh = plsc.VectorSubcoreMesh(core_axis_name="c", subcore_axis_name="s")

    @functools.partial(
        pl.kernel,
        mesh=mesh,
        compiler_params=pltpu.CompilerParams(
            use_tc_tiling_on_sc=True, needs_layout_passes=False),
        out_type=jax.ShapeDtypeStruct((D, B), jnp.float32),
        scratch_types=[pltpu.VMEM((b_per_w,), jnp.int32)]
        + [pltpu.VMEM((128, 128), jnp.float32)] * 2
        + [pltpu.VMEM((D, 128), jnp.float32)] * 2
        + [pltpu.SemaphoreType.DMA] * 4,
    )
    def k(rows, idx_hbm, outT, idx_v, vr0, vr1, vp0, vp1, g0, g1, w0, w1):
        vrows = (vr0, vr1)
        vpans = (vp0, vp1)
        gsems = (g0, g1)
        wsems = (w0, w1)
        wid = lax.axis_index("s") * _NUM_CORES + lax.axis_index("c")
        iota = lax.iota(jnp.int32, _L)
        rots = [(iota + d) % _L for d in range(_L)]
        pltpu.sync_copy(idx_hbm.at[pl.ds(wid * b_per_w, b_per_w)], idx_v)
        col0 = wid * n_pan

        def fire_gather(g, s):
            pltpu.async_copy(
                rows.at[idx_v.at[pl.ds(g * 128, 128)]], vrows[s], gsems[s])

        def drain_gather(s):
            pltpu.make_async_copy(
                rows.at[pl.ds(0, 128)], vrows[s], gsems[s]).wait()

        def fire_write(g, s):
            pltpu.async_copy(
                vpans[s],
                outT.at[:, pl.ds(pl.multiple_of((col0 + g) * 128, 128), 128)],
                wsems[s])

        def drain_write(s):
            pltpu.make_async_copy(
                vpans[s], outT.at[:, pl.ds(0, 128)], wsems[s]).wait()

        def assemble(s):
            def blk(i8, carry):
                ivec = i8 * _L + iota
                for c0 in range(0, D, _L):
                    cds = [c0 + rots[d] for d in range(_L)]
                    vals = [plsc.load_gather(vrows[s], [ivec, cd]) for cd in cds]
                    for cd, v in zip(cds, vals):
                        plsc.store_scatter(vpans[s], [cd, ivec], v)
                return carry
            lax.fori_loop(0, 8, blk, 0, unroll=2)

        fire_gather(0, 0)

        def visit(g, s):
            @pl.when(g + 1 < n_pan)
            def _():
                fire_gather(g + 1, 1 - s)
            drain_gather(s)

            @pl.when(g >= 2)
            def _():
                drain_write(s)
            assemble(s)
            fire_write(g, s)

        def body(i, carry):
            visit(2 * i, 0)
            visit(2 * i + 1, 1)
            return carry

        lax.fori_loop(0, n_pan // 2, body, 0, unroll=False)
        for s in (0, 1):
            drain_write(s)

    return k


def kernel(embedding, idx_mapping):
    B = idx_mapping.shape[0]
    N, D = embedding.shape
    tT = embedding.T
    tail = embedding[(N // 128) * 128:].reshape(-1)
    rows = _transpose_kernel(N, D)(tT, tail)
    outT = _gather_kernel(N, D, B)(rows, idx_mapping.astype(jnp.int32))
    return outT.T
